# Initial kernel scaffold; baseline (speedup 1.0000x reference)
import jax
import jax.numpy as jnp
from jax.experimental import pallas as pl


def kernel(ring_n, neighbor_corners):
    raise NotImplementedError("write your pallas kernel here")


# speedup vs baseline: 2.6913x; 2.6913x over previous
import functools

import jax
import jax.numpy as jnp
import numpy as np
from jax import lax
from jax.experimental import pallas as pl

NB = 32
NS = 32
BLOCK_F = 64
GATHER_GC = 8

_U32 = np.uint32
_TINY = np.float32(1.1754944e-38)

_ROT_A = (13, 15, 26, 6)
_ROT_B = (17, 29, 16, 24)


def _rotl(x, d):
    return (x << _U32(d)) | (x >> _U32(32 - d))


def _tf_rounds(x0, x1, rots):
    for r in rots:
        x0 = x0 + x1
        x1 = _rotl(x1, r) ^ x0
    return x0, x1


def _threefry2x32_xor(j):
    ks0 = _U32(0)
    ks1 = _U32(42)
    ks2 = ks0 ^ ks1 ^ _U32(0x1BD11BDA)
    x0 = jnp.zeros_like(j) + ks0
    x1 = j + ks1
    x0, x1 = _tf_rounds(x0, x1, _ROT_A)
    x0 = x0 + ks1
    x1 = x1 + (ks2 + _U32(1))
    x0, x1 = _tf_rounds(x0, x1, _ROT_B)
    x0 = x0 + ks2
    x1 = x1 + (ks0 + _U32(2))
    x0, x1 = _tf_rounds(x0, x1, _ROT_A)
    x0 = x0 + ks0
    x1 = x1 + (ks1 + _U32(3))
    x0, x1 = _tf_rounds(x0, x1, _ROT_B)
    x0 = x0 + ks1
    x1 = x1 + (ks2 + _U32(4))
    x0, x1 = _tf_rounds(x0, x1, _ROT_A)
    x0 = x0 + ks2
    x1 = x1 + (ks0 + _U32(5))
    return x0 ^ x1


def _body(num_faces, F, GC, c_ref, idx_ref, gath_ref):
    m = pl.program_id(0)
    fb = pl.program_id(1)

    c = c_ref[0]

    def dist(a, b):
        return jnp.sqrt(
            (c[:, :, a] - c[:, :, b]) ** 2
            + (c[:, :, a + 1] - c[:, :, b + 1]) ** 2
            + (c[:, :, a + 2] - c[:, :, b + 2]) ** 2
        )

    perim = dist(0, 3) + dist(3, 6) + dist(6, 0)
    invp = 1.0 / (perim + np.float32(1e-9))
    invp_t = invp.T
    invp_cols = jnp.broadcast_to(invp_t[:, :, None], (NB, F, NS)).reshape(NB, F * NS)

    col = lax.broadcasted_iota(jnp.int32, (NB, F * NS), 1)
    row = lax.broadcasted_iota(jnp.int32, (NB, F * NS), 0)
    base = (m * num_faces + fb * F) * (NS * NB)
    j = (base + col * NB + row).astype(_U32)

    bits = _threefry2x32_xor(j)
    fl = lax.bitcast_convert_type((bits >> _U32(9)) | _U32(0x3F800000), jnp.float32)
    fl = fl - 1.0
    u = jnp.where(fl == 0.0, _TINY, fl)
    w = -jnp.log(u) * invp_cols

    minv = jnp.min(w, axis=0, keepdims=True)
    idx = jnp.min(jnp.where(w == minv, row, np.int32(NB - 1)), axis=0, keepdims=True)
    idx_ref[0, 0] = idx.astype(jnp.float32)

    cf = c.reshape(F * NB, 9)
    for fg in range(F // GC):
        gsl = idx[0:1, fg * GC * NS:(fg + 1) * GC * NS]
        rcol = lax.broadcasted_iota(jnp.int32, (GC * NB, GC * NS), 1)
        g = gsl + (rcol & np.int32(~(NB - 1)))
        oht = (g == lax.broadcasted_iota(jnp.int32, (GC * NB, GC * NS), 0))
        oht = oht.astype(jnp.float32)
        src = cf[fg * GC * NB:(fg + 1) * GC * NB]
        res = lax.dot_general(oht, src, (((0,), (0,)), ((), ())),
                              precision=lax.Precision.HIGHEST,
                              preferred_element_type=jnp.float32)
        gath_ref[0, fg * GC * NS:(fg + 1) * GC * NS] = res


def kernel(ring_n, neighbor_corners):
    num_meshes, num_faces, nb = ring_n.shape
    c9 = neighbor_corners.reshape(num_meshes, num_faces, nb, 9)
    f = min(BLOCK_F, num_faces)
    gc = min(GATHER_GC, f)

    grid = (num_meshes, num_faces // f)
    idxf, gath = pl.pallas_call(
        functools.partial(_body, num_faces, f, gc),
        grid=grid,
        in_specs=[
            pl.BlockSpec((1, f, nb, 9), lambda m, i: (m, i, 0, 0)),
        ],
        out_specs=[
            pl.BlockSpec((1, 1, 1, f * NS), lambda m, i: (m, i, 0, 0)),
            pl.BlockSpec((1, f * NS, 9), lambda m, i: (m, i, 0)),
        ],
        out_shape=[
            jax.ShapeDtypeStruct((num_meshes, num_faces // f, 1, f * NS),
                                 jnp.float32),
            jax.ShapeDtypeStruct((num_meshes, num_faces * NS, 9), jnp.float32),
        ],
    )(c9)

    idxf = idxf.reshape(num_meshes, num_faces, NS)
    gath = gath.reshape(num_meshes, num_faces, NS, 9)
    rs_ring_n = jnp.concatenate([idxf, ring_n], axis=2)
    rs_nc = jnp.concatenate([gath, c9], axis=2)
    rs_nc = rs_nc.reshape(num_meshes, num_faces, 2 * NS, 3, 3)
    return (rs_ring_n, rs_nc)


# speedup vs baseline: 16.2071x; 6.0221x over previous
import functools

import jax
import jax.numpy as jnp
import numpy as np
from jax import lax
from jax.experimental import pallas as pl

NB = 32
NS = 32
BLOCK_F = 128

_U32 = np.uint32
_TINY = np.float32(1.1754944e-38)

_ROT_A = (13, 15, 26, 6)
_ROT_B = (17, 29, 16, 24)


def _rotl(x, d):
    return (x << _U32(d)) | (x >> _U32(32 - d))


def _tf_rounds(x0, x1, rots):
    for r in rots:
        x0 = x0 + x1
        x1 = _rotl(x1, r) ^ x0
    return x0, x1


def _threefry2x32_xor(j):
    ks0 = _U32(0)
    ks1 = _U32(42)
    ks2 = ks0 ^ ks1 ^ _U32(0x1BD11BDA)
    x1 = j + ks1
    x0 = x1
    x1 = _rotl(x1, _ROT_A[0]) ^ x0
    x0, x1 = _tf_rounds(x0, x1, _ROT_A[1:])
    x0 = x0 + ks1
    x1 = x1 + (ks2 + _U32(1))
    x0, x1 = _tf_rounds(x0, x1, _ROT_B)
    x0 = x0 + ks2
    x1 = x1 + (ks0 + _U32(2))
    x0, x1 = _tf_rounds(x0, x1, _ROT_A)
    x0 = x0 + ks0
    x1 = x1 + (ks1 + _U32(3))
    x0, x1 = _tf_rounds(x0, x1, _ROT_B)
    x0 = x0 + ks1
    x1 = x1 + (ks2 + _U32(4))
    x0, x1 = _tf_rounds(x0, x1, _ROT_A)
    x0 = x0 + ks2
    x1 = x1 + (ks0 + _U32(5))
    return x0 ^ x1


def _body(num_faces, F, ct_ref, ring_ref, ring_out_ref, gath_ref):
    m = pl.program_id(0)
    fb = pl.program_id(1)

    cs = [[ct_ref[0, a, b] for b in range(3)] for a in range(3)]

    def dist(a, b):
        return jnp.sqrt(
            (cs[a][0] - cs[b][0]) ** 2
            + (cs[a][1] - cs[b][1]) ** 2
            + (cs[a][2] - cs[b][2]) ** 2
        )

    perim = dist(0, 1) + dist(1, 2) + dist(2, 0)
    invp = 1.0 / (perim + np.float32(1e-9))

    s_iota = lax.broadcasted_iota(jnp.int32, (NS, F), 0)
    f_iota = lax.broadcasted_iota(jnp.int32, (NS, F), 1)
    jbase = ((m * num_faces + fb * F + f_iota) * NS + s_iota) * NB

    minv = None
    idx = None
    for n in range(NB):
        j = (jbase + n).astype(_U32)
        bits = _threefry2x32_xor(j)
        fl = lax.bitcast_convert_type((bits >> _U32(9)) | _U32(0x3F800000),
                                      jnp.float32)
        fl = fl - 1.0
        u = jnp.where(fl == 0.0, _TINY, fl)
        w = -jnp.log(u) * invp[n]
        if n == 0:
            minv = w
            idx = jnp.zeros((NS, F), jnp.int32)
        else:
            lt = w < minv
            minv = jnp.where(lt, w, minv)
            idx = jnp.where(lt, np.int32(n), idx)

    ring_out_ref[0] = jnp.concatenate(
        [idx.astype(jnp.float32), ring_ref[0]], axis=0)

    acc = [[jnp.zeros((NS, F), jnp.float32) for _ in range(3)]
           for _ in range(3)]
    for n in range(NB):
        mask = idx == n
        for a in range(3):
            for b in range(3):
                src_row = cs[a][b][n:n + 1]
                acc[a][b] = jnp.where(mask, src_row, acc[a][b])
    for a in range(3):
        for b in range(3):
            gath_ref[0, a, b] = acc[a][b]


def kernel(ring_n, neighbor_corners):
    num_meshes, num_faces, nb = ring_n.shape
    ct = jnp.transpose(neighbor_corners, (0, 3, 4, 2, 1))
    ring_t = jnp.transpose(ring_n, (0, 2, 1))
    f = min(BLOCK_F, num_faces)

    grid = (num_meshes, num_faces // f)
    ring_out, gath = pl.pallas_call(
        functools.partial(_body, num_faces, f),
        grid=grid,
        in_specs=[
            pl.BlockSpec((1, 3, 3, nb, f), lambda m, i: (m, 0, 0, 0, i)),
            pl.BlockSpec((1, nb, f), lambda m, i: (m, 0, i)),
        ],
        out_specs=[
            pl.BlockSpec((1, NS + nb, f), lambda m, i: (m, 0, i)),
            pl.BlockSpec((1, 3, 3, NS, f), lambda m, i: (m, 0, 0, 0, i)),
        ],
        out_shape=[
            jax.ShapeDtypeStruct((num_meshes, NS + nb, num_faces),
                                 jnp.float32),
            jax.ShapeDtypeStruct((num_meshes, 3, 3, NS, num_faces),
                                 jnp.float32),
        ],
    )(ct, ring_t)

    rs_ring_n = jnp.transpose(ring_out, (0, 2, 1))
    g5 = jnp.transpose(gath, (0, 4, 3, 1, 2))
    rs_nc = jnp.concatenate([g5, neighbor_corners], axis=2)
    return (rs_ring_n, rs_nc)


# speedup vs baseline: 17.3815x; 1.0725x over previous
import functools

import jax
import jax.numpy as jnp
import numpy as np
from jax import lax
from jax.experimental import pallas as pl

NB = 32
NS = 32
BLOCK_F = 128

_U32 = np.uint32
_TINY = np.float32(1.1754944e-38)

_ROT_A = (13, 15, 26, 6)
_ROT_B = (17, 29, 16, 24)


def _rotl(x, d):
    return (x << _U32(d)) | (x >> _U32(32 - d))


def _tf_rounds(x0, x1, rots):
    for r in rots:
        x0 = x0 + x1
        x1 = _rotl(x1, r) ^ x0
    return x0, x1


def _threefry2x32_xor(j):
    ks0 = _U32(0)
    ks1 = _U32(42)
    ks2 = ks0 ^ ks1 ^ _U32(0x1BD11BDA)
    x1 = j + ks1
    x0 = x1
    x1 = _rotl(x1, _ROT_A[0]) ^ x0
    x0, x1 = _tf_rounds(x0, x1, _ROT_A[1:])
    x0 = x0 + ks1
    x1 = x1 + (ks2 + _U32(1))
    x0, x1 = _tf_rounds(x0, x1, _ROT_B)
    x0 = x0 + ks2
    x1 = x1 + (ks0 + _U32(2))
    x0, x1 = _tf_rounds(x0, x1, _ROT_A)
    x0 = x0 + ks0
    x1 = x1 + (ks1 + _U32(3))
    x0, x1 = _tf_rounds(x0, x1, _ROT_B)
    x0 = x0 + ks1
    x1 = x1 + (ks2 + _U32(4))
    x0, x1 = _tf_rounds(x0, x1, _ROT_A)
    x0 = x0 + ks2
    x1 = x1 + (ks0 + _U32(5))
    return x0 ^ x1


def _body(num_faces, F, ct_ref, ring_ref, ring_out_ref, gath_ref):
    m = pl.program_id(0)
    fb = pl.program_id(1)

    cs = [[ct_ref[0, a, b] for b in range(3)] for a in range(3)]

    def dist(a, b):
        return jnp.sqrt(
            (cs[a][0] - cs[b][0]) ** 2
            + (cs[a][1] - cs[b][1]) ** 2
            + (cs[a][2] - cs[b][2]) ** 2
        )

    perim = dist(0, 1) + dist(1, 2) + dist(2, 0)
    invp = 1.0 / (perim + np.float32(1e-9))

    s_iota = lax.broadcasted_iota(jnp.int32, (NS, F), 0)
    f_iota = lax.broadcasted_iota(jnp.int32, (NS, F), 1)
    jbase = ((m * num_faces + fb * F + f_iota) * NS + s_iota) * NB

    minv = None
    idx = None
    for n in range(NB):
        j = (jbase + n).astype(_U32)
        bits = _threefry2x32_xor(j)
        fl = lax.bitcast_convert_type((bits >> _U32(9)) | _U32(0x3F800000),
                                      jnp.float32)
        fl = fl - 1.0
        u = jnp.where(fl == 0.0, _TINY, fl)
        w = -jnp.log(u) * invp[n]
        if n == 0:
            minv = w
            idx = jnp.zeros((NS, F), jnp.int32)
        else:
            lt = w < minv
            minv = jnp.where(lt, w, minv)
            idx = jnp.where(lt, np.int32(n), idx)

    ring_out_ref[0] = jnp.concatenate(
        [idx.astype(jnp.float32), ring_ref[0]], axis=0)

    acc = [[jnp.zeros((NS, F), jnp.float32) for _ in range(3)]
           for _ in range(3)]
    for n in range(NB):
        mask = idx == n
        for a in range(3):
            for b in range(3):
                src_row = cs[a][b][n:n + 1]
                acc[a][b] = jnp.where(mask, src_row, acc[a][b])
    for a in range(3):
        for b in range(3):
            gath_ref[0, a, b] = jnp.concatenate([acc[a][b], cs[a][b]], axis=0)


def kernel(ring_n, neighbor_corners):
    num_meshes, num_faces, nb = ring_n.shape
    ct = jnp.transpose(neighbor_corners, (0, 3, 4, 2, 1))
    ring_t = jnp.transpose(ring_n, (0, 2, 1))
    f = min(BLOCK_F, num_faces)

    grid = (num_meshes, num_faces // f)
    ring_out, gath = pl.pallas_call(
        functools.partial(_body, num_faces, f),
        grid=grid,
        in_specs=[
            pl.BlockSpec((1, 3, 3, nb, f), lambda m, i: (m, 0, 0, 0, i)),
            pl.BlockSpec((1, nb, f), lambda m, i: (m, 0, i)),
        ],
        out_specs=[
            pl.BlockSpec((1, NS + nb, f), lambda m, i: (m, 0, i)),
            pl.BlockSpec((1, 3, 3, NS + nb, f), lambda m, i: (m, 0, 0, 0, i)),
        ],
        out_shape=[
            jax.ShapeDtypeStruct((num_meshes, NS + nb, num_faces),
                                 jnp.float32),
            jax.ShapeDtypeStruct((num_meshes, 3, 3, NS + nb, num_faces),
                                 jnp.float32),
        ],
    )(ct, ring_t)

    rs_ring_n = jnp.transpose(ring_out, (0, 2, 1))
    rs_nc = jnp.transpose(gath, (0, 4, 3, 1, 2))
    return (rs_ring_n, rs_nc)
